# baseline (speedup 1.0000x reference)
import functools

import jax
import jax.numpy as jnp
from jax import lax
from jax.experimental import pallas as pl
from jax.experimental.pallas import tpu as pltpu
from jax.experimental.pallas import tpu_sc as plsc

B = 16384
D = 64
CHUNK = 128
NC, NS = 2, 16
NW = NC * NS
B_PER_W = B // NW
K_PER_W = B_PER_W // CHUNK
NGRP = CHUNK // 16


def _sc_gather_body(ptab_hbm, atab_hbm, pidx_hbm, aidx_hbm,
                    pe_hbm, ae_hbm,
                    pidx_v, grp_v, aidx_v, rows_v, ae_v, sem, asem):
    wid = lax.axis_index("s") * NC + lax.axis_index("c")
    rowbase = wid * B_PER_W
    pltpu.sync_copy(pidx_hbm.at[pl.ds(wid * K_PER_W, K_PER_W)], pidx_v)
    pltpu.sync_copy(aidx_hbm.at[pl.ds(wid * K_PER_W, K_PER_W)], aidx_v)
    for j in range(K_PER_W):
        for k in range(NGRP):
            grp_v[j, pl.ds(k * 16, 16)] = pidx_v[j, pl.ds(k * 16, 16)] >> 1
    prev = None
    for j in range(K_PER_W):
        cur = pltpu.async_copy(
            ptab_hbm.at[grp_v.at[j]],
            rows_v.at[pl.ds(j * CHUNK, CHUNK)], sem)
        if prev is not None:
            prev.wait()
        prev = cur
    prev.wait()
    pltpu.sync_copy(rows_v, pe_hbm.at[pl.ds(rowbase, B_PER_W)])
    for j in range(K_PER_W):
        pltpu.async_copy(atab_hbm.at[aidx_v.at[j]], ae_v, asem).wait()
        pltpu.sync_copy(ae_v, ae_hbm.at[pl.ds(rowbase + j * CHUNK, CHUNK)])


@jax.jit
def _sc_gather(product_pairs, action_type_table, product_ids, action_types):
    atab_p = jnp.pad(action_type_table, ((0, 0), (0, 64)))
    pidx = product_ids.reshape(B // CHUNK, CHUNK)
    aidx = action_types.reshape(B // CHUNK, CHUNK)
    mesh = plsc.VectorSubcoreMesh(core_axis_name="c", subcore_axis_name="s")
    out_t = (jax.ShapeDtypeStruct((B, 128), jnp.float32),
             jax.ShapeDtypeStruct((B, 128), jnp.float32))
    fn = pl.kernel(
        _sc_gather_body,
        mesh=mesh,
        out_type=out_t,
        scratch_types=[
            pltpu.VMEM((K_PER_W, CHUNK), jnp.int32),
            pltpu.VMEM((K_PER_W, CHUNK), jnp.int32),
            pltpu.VMEM((K_PER_W, CHUNK), jnp.int32),
            pltpu.VMEM((B_PER_W, 128), jnp.float32),
            pltpu.VMEM((CHUNK, 128), jnp.float32),
            pltpu.SemaphoreType.DMA,
            pltpu.SemaphoreType.DMA,
        ],
    )
    return fn(product_pairs, atab_p, pidx, aidx)


def _dense_body(ids_ref, ae_ref, pp_ref, tf_ref, cf_ref,
                tw_ref, tb_ref, cw_ref, cb_ref,
                wa_ref, wp0_ref, wp1_ref, wt_ref, wc_ref, fb_ref, out_ref):
    f32 = jnp.float32
    cdims = (((1,), (1,)), ((), ()))
    t_emb = lax.dot_general(tf_ref[...], tw_ref[...], cdims,
                            preferred_element_type=f32) + tb_ref[...]
    c_emb = lax.dot_general(cf_ref[...], cw_ref[...], cdims,
                            preferred_element_type=f32) + cb_ref[...]
    mm = (((1,), (0,)), ((), ()))
    acc = lax.dot_general(ae_ref[...], wa_ref[...], mm, preferred_element_type=f32)
    acc += lax.dot_general(t_emb, wt_ref[...], mm, preferred_element_type=f32)
    acc += lax.dot_general(c_emb, wc_ref[...], mm, preferred_element_type=f32)
    pp = pp_ref[...]
    d0 = lax.dot_general(pp, wp0_ref[...], mm, preferred_element_type=f32)
    d1 = lax.dot_general(pp, wp1_ref[...], mm, preferred_element_type=f32)
    par_lane = (ids_ref[0, 0, :] & 1).astype(f32).reshape(1, -1)
    par_col = lax.dot_general(par_lane, jnp.ones((1, 1), f32),
                              (((0,), (0,)), ((), ())),
                              preferred_element_type=f32)
    acc += d0 + (d1 - d0) * par_col
    out_ref[...] = jnp.maximum(acc + fb_ref[...], 0.0)


@functools.partial(jax.jit, static_argnames=("blk",))
def _tc_dense(product_ids3, action_emb, product_pairs,
              temporal_features, context_features,
              temporal_W, temporal_b, context_W, context_b,
              wa, wp0, wp1, wt, wc, fb, blk=2048):
    grid = (B // blk,)
    row_spec = lambda d: pl.BlockSpec((blk, d), lambda i: (i, 0))
    full = lambda a: pl.BlockSpec(a.shape, lambda i: (0,) * a.ndim)
    return pl.pallas_call(
        _dense_body,
        grid=grid,
        in_specs=[
            pl.BlockSpec((1, 1, blk), lambda i: (i, 0, 0)),
            row_spec(128), row_spec(128), row_spec(5), row_spec(10),
            full(temporal_W), full(temporal_b), full(context_W), full(context_b),
            full(wa), full(wp0), full(wp1), full(wt), full(wc), full(fb),
        ],
        out_specs=pl.BlockSpec((blk, 128), lambda i: (i, 0)),
        out_shape=jax.ShapeDtypeStruct((B, 128), jnp.float32),
    )(product_ids3, action_emb, product_pairs,
      temporal_features, context_features,
      temporal_W, temporal_b, context_W, context_b,
      wa, wp0, wp1, wt, wc, fb)


def kernel(action_types, product_ids, temporal_features, context_features,
           action_type_table, product_table,
           temporal_W, temporal_b, context_W, context_b,
           fusion_W, fusion_b):
    pairs = product_table.reshape(500000, 128)
    pp, ae_w = _sc_gather(pairs, action_type_table,
                          product_ids, action_types)
    zeros64 = jnp.zeros((64, 128), jnp.float32)
    wp = fusion_W[:, 64:128].T
    wa_p = jnp.concatenate([fusion_W[:, 0:64].T, zeros64], axis=0)
    wp0 = jnp.concatenate([wp, zeros64], axis=0)
    wp1 = jnp.concatenate([zeros64, wp], axis=0)
    wt = fusion_W[:, 128:160].T
    wc = fusion_W[:, 160:192].T
    ids3 = product_ids.reshape(B // 2048, 1, 2048)
    return _tc_dense(ids3, ae_w, pp, temporal_features, context_features,
                     temporal_W, temporal_b.reshape(1, 32),
                     context_W, context_b.reshape(1, 32),
                     wa_p, wp0, wp1, wt, wc, fusion_b.reshape(1, 128))
